# baseline (device time: 211903 ns/iter reference)
import jax
import jax.numpy as jnp
from jax import lax
from jax.experimental import pallas as pl
from jax.experimental.pallas import tpu as pltpu

N_DEV = 4
EPS = 1e-5
SCALE = 0.08838834764831843
BF16 = jnp.bfloat16
F32 = jnp.float32


def _vmem(n=1):
    return [pl.BlockSpec(memory_space=pltpu.VMEM)] * n


def _matmul(a, w, out_dtype=BF16):
    M = a.shape[0]
    N = w.shape[1]

    def body(a_ref, w_ref, o_ref):
        o = jnp.dot(a_ref[:, :].astype(BF16), w_ref[:, :].astype(BF16),
                    preferred_element_type=F32)
        o_ref[:, :] = o.astype(out_dtype)

    return pl.pallas_call(
        body,
        out_shape=jax.ShapeDtypeStruct((M, N), out_dtype),
        in_specs=_vmem(2),
        out_specs=_vmem()[0],
    )(a, w)


def _layernorm(h):
    m = jnp.mean(h, axis=1, keepdims=True)
    c = h - m
    var = jnp.mean(c * c, axis=1, keepdims=True)
    return c * lax.rsqrt(var + EPS)


def _mod_ln(x, t_emb, W_mod, B, S, D):

    def body(x_ref, temb_ref, wmod_ref, mod_ref, xm_ref):
        mod = jnp.dot(temb_ref[:, :], wmod_ref[:, :],
                      preferred_element_type=F32)
        mod_ref[:, :] = mod
        for b in range(B):
            xln = _layernorm(x_ref[b, :, :])
            xm_ref[b * S:(b + 1) * S, :] = (
                xln * (1.0 + mod[b, 0:D][None, :])
                + mod[b, D:2 * D][None, :]).astype(BF16)

    return pl.pallas_call(
        body,
        out_shape=[
            jax.ShapeDtypeStruct((B, 6 * D), F32),
            jax.ShapeDtypeStruct((B * S, D), BF16),
        ],
        in_specs=_vmem(3),
        out_specs=_vmem(2),
    )(x, t_emb, W_mod)


def _attention_wo(q, k, v, Wo, B, S, H_loc, Dh, D):

    def body(q_ref, k_ref, v_ref, wo_ref, p1_ref):
        h = pl.program_id(0) % H_loc
        s_ = lax.dot_general(q_ref[:, :], k_ref[:, :],
                             (((1,), (1,)), ((), ())),
                             preferred_element_type=F32) * SCALE
        mx = jnp.max(s_, axis=1, keepdims=True)
        p = (s_ - mx).astype(BF16)
        p = jnp.exp(p)
        l = jnp.sum(p, axis=1, keepdims=True, dtype=F32)
        o = jnp.dot(p, v_ref[:, :], preferred_element_type=F32) / l
        contrib = jnp.dot(o.astype(BF16), wo_ref[:, :].astype(BF16),
                          preferred_element_type=F32).astype(BF16)

        @pl.when(h == 0)
        def _():
            p1_ref[:, :] = contrib

        @pl.when(h != 0)
        def _():
            p1_ref[:, :] = p1_ref[:, :] + contrib

    spec = pl.BlockSpec((S, Dh), lambda i: (i // H_loc, i % H_loc))
    return pl.pallas_call(
        body,
        grid=(B * H_loc,),
        out_shape=jax.ShapeDtypeStruct((B * S, D), BF16),
        in_specs=[spec, spec, spec,
                  pl.BlockSpec((Dh, D), lambda i: (i % H_loc, 0))],
        out_specs=pl.BlockSpec((S, D), lambda i: (i // H_loc, 0)),
    )(q, k, v, Wo)


def _ring_all_reduce(my, partial_ref, acc, comm, send_sems, recv_sems):
    R = acc.shape[0]
    H = R // 2
    Ch = H // N_DEV
    right = jnp.mod(my + 1, N_DEV)
    left = jnp.mod(my + 3, N_DEV)

    def copy(src, s_off, dst, d_off, sem, tgt):
        return pltpu.make_async_remote_copy(
            src_ref=src.at[pl.ds(s_off, Ch), :],
            dst_ref=dst if d_off is None else dst.at[pl.ds(d_off, Ch), :],
            send_sem=send_sems.at[sem],
            recv_sem=recv_sems.at[sem],
            device_id=(tgt,),
            device_id_type=pl.DeviceIdType.MESH,
        )

    for s in range(N_DEV - 1):
        sc0 = jnp.mod(my - s + 8, N_DEV)
        rc0 = jnp.mod(my - s - 1 + 8, N_DEV)
        sc1 = jnp.mod(my + s, N_DEV)
        rc1 = jnp.mod(my + s + 1, N_DEV)
        src = partial_ref if s == 0 else acc
        r0 = copy(src, sc0 * Ch, comm.at[s], None, s, right)
        r1 = copy(src, H + sc1 * Ch, comm.at[3 + s], None, 3 + s, left)
        r0.start()
        r1.start()
        r0.wait()
        r1.wait()
        acc[pl.ds(rc0 * Ch, Ch), :] = (
            partial_ref[pl.ds(rc0 * Ch, Ch), :] + comm[s])
        acc[pl.ds(H + rc1 * Ch, Ch), :] = (
            partial_ref[pl.ds(H + rc1 * Ch, Ch), :] + comm[3 + s])
    for t in range(N_DEV - 1):
        sc0 = jnp.mod(my + 1 - t + 8, N_DEV)
        sc1 = jnp.mod(my - 1 + t + 8, N_DEV)
        r0 = copy(acc, sc0 * Ch, acc, sc0 * Ch, 6 + t, right)
        r1 = copy(acc, H + sc1 * Ch, acc, H + sc1 * Ch, 9 + t, left)
        r0.start()
        r1.start()
        r0.wait()
        r1.wait()


def _barriers(bar_sems):
    my = lax.axis_index("i")
    right = jnp.mod(my + 1, N_DEV)
    left = jnp.mod(my + 3, N_DEV)

    def entry():
        bsem = pltpu.get_barrier_semaphore()
        for nbr in (left, right):
            pl.semaphore_signal(bsem, inc=1, device_id=(nbr,),
                                device_id_type=pl.DeviceIdType.MESH)
        pl.semaphore_wait(bsem, 2)

    def exit_():
        for nbr in (left, right):
            pl.semaphore_signal(bar_sems.at[0], inc=1, device_id=(nbr,),
                                device_id_type=pl.DeviceIdType.MESH)
        pl.semaphore_wait(bar_sems.at[0], 2)

    return my, entry, exit_


def _ar_scratch(R, D):
    Ch = R // 2 // N_DEV
    return [
        pltpu.VMEM((R, D), BF16),
        pltpu.VMEM((6, Ch, D), BF16),
        pltpu.SemaphoreType.DMA((12,)),
        pltpu.SemaphoreType.DMA((12,)),
        pltpu.SemaphoreType.REGULAR((1,)),
    ]


def _ar1_residual_ln(p1, x, mod, B, S, D):

    def body(p1_ref, x_ref, mod_ref, x1_ref, xln2_ref,
             acc, comm, send_sems, recv_sems, bar_sems):
        my, entry, exit_ = _barriers(bar_sems)
        entry()
        _ring_all_reduce(my, p1_ref, acc, comm, send_sems, recv_sems)
        n_chunks = 8
        rows = B * S // n_chunks
        per_b = S // rows
        for c in range(n_chunks):
            b = c // per_b
            rb = pl.ds((c % per_b) * rows, rows)
            r = pl.ds(c * rows, rows)
            x1 = (x_ref[b, rb, :]
                  + mod_ref[b, 2 * D:3 * D][None, :] * acc[r, :].astype(F32))
            x1_ref[r, :] = x1
            xln2_ref[r, :] = (
                _layernorm(x1) * (1.0 + mod_ref[b, 3 * D:4 * D][None, :])
                + mod_ref[b, 4 * D:5 * D][None, :]).astype(BF16)
        exit_()

    return pl.pallas_call(
        body,
        out_shape=[
            jax.ShapeDtypeStruct((B * S, D), F32),
            jax.ShapeDtypeStruct((B * S, D), BF16),
        ],
        in_specs=_vmem(3),
        out_specs=_vmem(2),
        scratch_shapes=_ar_scratch(B * S, D),
        compiler_params=pltpu.CompilerParams(
            collective_id=1, vmem_limit_bytes=40 * 1024 * 1024),
    )(p1, x, mod)


def _ar2_residual(p2, x1, mod, B, S, D):

    def body(p2_ref, x1_ref, mod_ref, out_ref,
             acc, comm, send_sems, recv_sems, bar_sems):
        my, entry, exit_ = _barriers(bar_sems)
        entry()
        _ring_all_reduce(my, p2_ref, acc, comm, send_sems, recv_sems)
        n_chunks = 8
        rows = B * S // n_chunks
        per_b = S // rows
        for c in range(n_chunks):
            b = c // per_b
            rb = pl.ds((c % per_b) * rows, rows)
            r = pl.ds(c * rows, rows)
            out_ref[b, rb, :] = (x1_ref[r, :]
                                 + mod_ref[b, 5 * D:6 * D][None, :]
                                 * acc[r, :].astype(F32))
        exit_()

    return pl.pallas_call(
        body,
        out_shape=jax.ShapeDtypeStruct((B, S, D), F32),
        in_specs=_vmem(3),
        out_specs=_vmem()[0],
        scratch_shapes=_ar_scratch(B * S, D),
        compiler_params=pltpu.CompilerParams(
            collective_id=2, vmem_limit_bytes=40 * 1024 * 1024),
    )(p2, x1, mod)


def _ffn(xln2, W_ff1, W_ff2):

    def body(a_ref, w1_ref, w2_ref, o_ref):
        h = jnp.dot(a_ref[:, :], w1_ref[:, :].astype(BF16),
                    preferred_element_type=F32)
        h = (h / (1.0 + jnp.exp(-h))).astype(BF16)
        o_ref[:, :] = jnp.dot(h, w2_ref[:, :].astype(BF16),
                              preferred_element_type=F32).astype(BF16)

    return pl.pallas_call(
        body,
        out_shape=jax.ShapeDtypeStruct((xln2.shape[0], W_ff2.shape[1]), BF16),
        in_specs=_vmem(3),
        out_specs=_vmem()[0],
    )(xln2, W_ff1, W_ff2)


def kernel(x, Wq, Wk, Wv, Wo, t_emb, W_mod, W_ff1, W_ff2):
    B, S, D = x.shape
    Dh = 128
    H_loc = Wq.shape[1] // Dh

    mod, xm = _mod_ln(x, t_emb, W_mod, B, S, D)
    q = _matmul(xm, Wq)
    k = _matmul(xm, Wk)
    v = _matmul(xm, Wv)
    p1 = _attention_wo(q, k, v, Wo, B, S, H_loc, Dh, D)
    x1, xln2 = _ar1_residual_ln(p1, x, mod, B, S, D)
    p2 = _ffn(xln2, W_ff1, W_ff2)
    return _ar2_residual(p2, x1, mod, B, S, D)


# device time: 177773 ns/iter; 1.1920x vs baseline; 1.1920x over previous
import jax
import jax.numpy as jnp
from jax import lax
from jax.experimental import pallas as pl
from jax.experimental.pallas import tpu as pltpu

N_DEV = 4
EPS = 1e-5
SCALE = 0.08838834764831843
BF16 = jnp.bfloat16
F32 = jnp.float32


def _vmem(n=1):
    return [pl.BlockSpec(memory_space=pltpu.VMEM)] * n


def _matmul(a, w, out_dtype=BF16):
    M = a.shape[0]
    N = w.shape[1]

    def body(a_ref, w_ref, o_ref):
        o = jnp.dot(a_ref[:, :].astype(BF16), w_ref[:, :].astype(BF16),
                    preferred_element_type=F32)
        o_ref[:, :] = o.astype(out_dtype)

    return pl.pallas_call(
        body,
        out_shape=jax.ShapeDtypeStruct((M, N), out_dtype),
        in_specs=_vmem(2),
        out_specs=_vmem()[0],
    )(a, w)


def _layernorm(h):
    m = jnp.mean(h, axis=1, keepdims=True)
    c = h - m
    var = jnp.mean(c * c, axis=1, keepdims=True)
    return c * lax.rsqrt(var + EPS)


def _mod_ln(x, t_emb, W_mod, B, S, D):

    def body(x_ref, temb_ref, wmod_ref, mod_ref, xm_ref):
        mod = jnp.dot(temb_ref[:, :], wmod_ref[:, :],
                      preferred_element_type=F32)
        mod_ref[:, :] = mod
        for b in range(B):
            xln = _layernorm(x_ref[b, :, :])
            xm_ref[b * S:(b + 1) * S, :] = (
                xln * (1.0 + mod[b, 0:D][None, :])
                + mod[b, D:2 * D][None, :]).astype(BF16)

    return pl.pallas_call(
        body,
        out_shape=[
            jax.ShapeDtypeStruct((B, 6 * D), F32),
            jax.ShapeDtypeStruct((B * S, D), BF16),
        ],
        in_specs=_vmem(3),
        out_specs=_vmem(2),
    )(x, t_emb, W_mod)


def _qkv(xm, Wq, Wk, Wv):
    M, D = xm.shape

    def body(a_ref, wq_ref, wk_ref, wv_ref, q_ref, k_ref, v_ref):
        a = a_ref[:, :]
        q_ref[:, :] = (jnp.dot(a, wq_ref[:, :].astype(BF16),
                               preferred_element_type=F32) * SCALE).astype(BF16)
        k_ref[:, :] = jnp.dot(a, wk_ref[:, :].astype(BF16),
                              preferred_element_type=F32).astype(BF16)
        v_ref[:, :] = jnp.dot(a, wv_ref[:, :].astype(BF16),
                              preferred_element_type=F32).astype(BF16)

    sh = jax.ShapeDtypeStruct((M, Wq.shape[1]), BF16)
    return pl.pallas_call(
        body,
        out_shape=[sh, sh, sh],
        in_specs=_vmem(4),
        out_specs=_vmem(3),
    )(xm, Wq, Wk, Wv)


def _attention(q, k, v, B, S, H_loc, Dh):

    def body(q_ref, k_ref, v_ref, o_ref):
        s_ = lax.dot_general(q_ref[:, :], k_ref[:, :],
                             (((1,), (1,)), ((), ())),
                             preferred_element_type=F32)
        p = jnp.exp(s_.astype(BF16))
        l = jnp.sum(p, axis=1, keepdims=True, dtype=F32)
        o = jnp.dot(p, v_ref[:, :], preferred_element_type=F32) / l
        o_ref[:, :] = o.astype(BF16)

    spec = pl.BlockSpec((S, Dh), lambda i: (i // H_loc, i % H_loc))
    return pl.pallas_call(
        body,
        grid=(B * H_loc,),
        out_shape=jax.ShapeDtypeStruct((B * S, H_loc * Dh), BF16),
        in_specs=[spec, spec, spec],
        out_specs=spec,
    )(q, k, v)


def _ring_all_reduce(my, partial_ref, acc, comm, send_sems, recv_sems,
                     ep=None):
    R = acc.shape[0]
    H = R // 2
    Ch = H // N_DEV
    right = jnp.mod(my + 1, N_DEV)
    left = jnp.mod(my + 3, N_DEV)

    def copy(src, s_off, dst, d_off, sem, tgt):
        return pltpu.make_async_remote_copy(
            src_ref=src.at[pl.ds(s_off, Ch), :],
            dst_ref=dst if d_off is None else dst.at[pl.ds(d_off, Ch), :],
            send_sem=send_sems.at[sem],
            recv_sem=recv_sems.at[sem],
            device_id=(tgt,),
            device_id_type=pl.DeviceIdType.MESH,
        )

    for s in range(N_DEV - 1):
        sc0 = jnp.mod(my - s + 8, N_DEV)
        rc0 = jnp.mod(my - s - 1 + 8, N_DEV)
        sc1 = jnp.mod(my + s, N_DEV)
        rc1 = jnp.mod(my + s + 1, N_DEV)
        src = partial_ref if s == 0 else acc
        r0 = copy(src, sc0 * Ch, comm.at[s], None, s, right)
        r1 = copy(src, H + sc1 * Ch, comm.at[3 + s], None, 3 + s, left)
        r0.start()
        r1.start()
        r0.wait()
        r1.wait()
        acc[pl.ds(rc0 * Ch, Ch), :] = (
            partial_ref[pl.ds(rc0 * Ch, Ch), :] + comm[s])
        acc[pl.ds(H + rc1 * Ch, Ch), :] = (
            partial_ref[pl.ds(H + rc1 * Ch, Ch), :] + comm[3 + s])
    for t in range(N_DEV - 1):
        sc0 = jnp.mod(my + 1 - t + 8, N_DEV)
        sc1 = jnp.mod(my - 1 + t + 8, N_DEV)
        r0 = copy(acc, sc0 * Ch, acc, sc0 * Ch, 6 + t, right)
        r1 = copy(acc, H + sc1 * Ch, acc, H + sc1 * Ch, 9 + t, left)
        r0.start()
        r1.start()
        if ep is not None:
            ep(sc0, sc1)
        r0.wait()
        r1.wait()
    if ep is not None:
        ep(jnp.mod(my - 2 + 8, N_DEV), jnp.mod(my + 2, N_DEV))


def _barriers(bar_sems):
    my = lax.axis_index("i")
    right = jnp.mod(my + 1, N_DEV)
    left = jnp.mod(my + 3, N_DEV)

    def entry():
        bsem = pltpu.get_barrier_semaphore()
        for nbr in (left, right):
            pl.semaphore_signal(bsem, inc=1, device_id=(nbr,),
                                device_id_type=pl.DeviceIdType.MESH)
        pl.semaphore_wait(bsem, 2)

    def exit_():
        for nbr in (left, right):
            pl.semaphore_signal(bar_sems.at[0], inc=1, device_id=(nbr,),
                                device_id_type=pl.DeviceIdType.MESH)
        pl.semaphore_wait(bar_sems.at[0], 2)

    return my, entry, exit_


def _ar_scratch(R, D):
    Ch = R // 2 // N_DEV
    return [
        pltpu.VMEM((R, D), BF16),
        pltpu.VMEM((6, Ch, D), BF16),
        pltpu.SemaphoreType.DMA((12,)),
        pltpu.SemaphoreType.DMA((12,)),
        pltpu.SemaphoreType.REGULAR((1,)),
    ]


def _ar1_residual_ln(p1, x, mod, B, S, D):

    Ch = S // N_DEV

    def body(p1_ref, x_ref, mod_ref, x1_ref, xln2_ref,
             acc, comm, send_sems, recv_sems, bar_sems):
        my, entry, exit_ = _barriers(bar_sems)
        entry()

        def ep(ck_cw, ck_ccw):
            for b, ck in ((0, ck_cw), (1, ck_ccw)):
                rb = pl.ds(ck * Ch, Ch)
                r = pl.ds(b * S + ck * Ch, Ch)
                x1 = (x_ref[b, rb, :]
                      + mod_ref[b, 2 * D:3 * D][None, :]
                      * acc[r, :].astype(F32))
                x1_ref[r, :] = x1
                xln2_ref[r, :] = (
                    _layernorm(x1) * (1.0 + mod_ref[b, 3 * D:4 * D][None, :])
                    + mod_ref[b, 4 * D:5 * D][None, :]).astype(BF16)

        _ring_all_reduce(my, p1_ref, acc, comm, send_sems, recv_sems, ep=ep)
        exit_()

    return pl.pallas_call(
        body,
        out_shape=[
            jax.ShapeDtypeStruct((B * S, D), F32),
            jax.ShapeDtypeStruct((B * S, D), BF16),
        ],
        in_specs=_vmem(3),
        out_specs=_vmem(2),
        scratch_shapes=_ar_scratch(B * S, D),
        compiler_params=pltpu.CompilerParams(
            collective_id=1, vmem_limit_bytes=40 * 1024 * 1024),
    )(p1, x, mod)


def _ar2_residual(p2, x1, mod, B, S, D):

    Ch = S // N_DEV

    def body(p2_ref, x1_ref, mod_ref, out_ref,
             acc, comm, send_sems, recv_sems, bar_sems):
        my, entry, exit_ = _barriers(bar_sems)
        entry()

        def ep(ck_cw, ck_ccw):
            for b, ck in ((0, ck_cw), (1, ck_ccw)):
                rb = pl.ds(ck * Ch, Ch)
                r = pl.ds(b * S + ck * Ch, Ch)
                out_ref[b, rb, :] = (x1_ref[r, :]
                                     + mod_ref[b, 5 * D:6 * D][None, :]
                                     * acc[r, :].astype(F32))

        _ring_all_reduce(my, p2_ref, acc, comm, send_sems, recv_sems, ep=ep)
        exit_()

    return pl.pallas_call(
        body,
        out_shape=jax.ShapeDtypeStruct((B, S, D), F32),
        in_specs=_vmem(3),
        out_specs=_vmem()[0],
        scratch_shapes=_ar_scratch(B * S, D),
        compiler_params=pltpu.CompilerParams(
            collective_id=2, vmem_limit_bytes=40 * 1024 * 1024),
    )(p2, x1, mod)


def _ffn(xln2, W_ff1, W_ff2):

    def body(a_ref, w1_ref, w2_ref, o_ref):
        h = jnp.dot(a_ref[:, :], w1_ref[:, :].astype(BF16),
                    preferred_element_type=F32)
        h = (h / (1.0 + jnp.exp(-h))).astype(BF16)
        o_ref[:, :] = jnp.dot(h, w2_ref[:, :].astype(BF16),
                              preferred_element_type=F32).astype(BF16)

    return pl.pallas_call(
        body,
        out_shape=jax.ShapeDtypeStruct((xln2.shape[0], W_ff2.shape[1]), BF16),
        in_specs=_vmem(3),
        out_specs=_vmem()[0],
    )(xln2, W_ff1, W_ff2)


def kernel(x, Wq, Wk, Wv, Wo, t_emb, W_mod, W_ff1, W_ff2):
    B, S, D = x.shape
    Dh = 128
    H_loc = Wq.shape[1] // Dh

    mod, xm = _mod_ln(x, t_emb, W_mod, B, S, D)
    q, k, v = _qkv(xm, Wq, Wk, Wv)
    ao = _attention(q, k, v, B, S, H_loc, Dh)
    p1 = _matmul(ao, Wo)
    x1, xln2 = _ar1_residual_ln(p1, x, mod, B, S, D)
    p2 = _ffn(xln2, W_ff1, W_ff2)
    return _ar2_residual(p2, x1, mod, B, S, D)


# device time: 165921 ns/iter; 1.2771x vs baseline; 1.0714x over previous
import jax
import jax.numpy as jnp
from jax import lax
from jax.experimental import pallas as pl
from jax.experimental.pallas import tpu as pltpu

N_DEV = 4
EPS = 1e-5
SCALE = 0.08838834764831843
BF16 = jnp.bfloat16
F32 = jnp.float32


def _vmem(n=1):
    return [pl.BlockSpec(memory_space=pltpu.VMEM)] * n


def _matmul(a, w, out_dtype=BF16):
    M = a.shape[0]
    N = w.shape[1]

    def body(a_ref, w_ref, o_ref):
        o = jnp.dot(a_ref[:, :].astype(BF16), w_ref[:, :].astype(BF16),
                    preferred_element_type=F32)
        o_ref[:, :] = o.astype(out_dtype)

    return pl.pallas_call(
        body,
        out_shape=jax.ShapeDtypeStruct((M, N), out_dtype),
        in_specs=_vmem(2),
        out_specs=_vmem()[0],
    )(a, w)


def _layernorm(h):
    m = jnp.mean(h, axis=1, keepdims=True)
    c = h - m
    var = jnp.mean(c * c, axis=1, keepdims=True)
    return c * lax.rsqrt(var + EPS)


def _mod_ln(x, t_emb, W_mod, B, S, D):

    def body(x_ref, temb_ref, wmod_ref, mod_ref, xm_ref, xbf_ref):
        mod = jnp.dot(temb_ref[:, :], wmod_ref[:, :],
                      preferred_element_type=F32)
        mod_ref[:, :] = mod
        for b in range(B):
            xv = x_ref[b, :, :]
            xbf_ref[b, :, :] = xv.astype(BF16)
            xln = _layernorm(xv)
            xm_ref[b * S:(b + 1) * S, :] = (
                xln * (1.0 + mod[b, 0:D][None, :])
                + mod[b, D:2 * D][None, :]).astype(BF16)

    return pl.pallas_call(
        body,
        out_shape=[
            jax.ShapeDtypeStruct((B, 6 * D), F32),
            jax.ShapeDtypeStruct((B * S, D), BF16),
            jax.ShapeDtypeStruct((B, S, D), BF16),
        ],
        in_specs=_vmem(3),
        out_specs=_vmem(3),
    )(x, t_emb, W_mod)


def _qkv(xm, Wq, Wk, Wv):
    M, D = xm.shape

    def body(a_ref, wq_ref, wk_ref, wv_ref, q_ref, k_ref, v_ref):
        a = a_ref[:, :]
        q_ref[:, :] = (jnp.dot(a, wq_ref[:, :].astype(BF16),
                               preferred_element_type=F32) * SCALE).astype(BF16)
        k_ref[:, :] = jnp.dot(a, wk_ref[:, :].astype(BF16),
                              preferred_element_type=F32).astype(BF16)
        v_ref[:, :] = jnp.dot(a, wv_ref[:, :].astype(BF16),
                              preferred_element_type=F32).astype(BF16)

    sh = jax.ShapeDtypeStruct((M, Wq.shape[1]), BF16)
    return pl.pallas_call(
        body,
        out_shape=[sh, sh, sh],
        in_specs=_vmem(4),
        out_specs=_vmem(3),
    )(xm, Wq, Wk, Wv)


def _attention(q, k, v, B, S, H_loc, Dh):

    def body(q_ref, k_ref, v_ref, o_ref):
        s_ = lax.dot_general(q_ref[:, :], k_ref[:, :],
                             (((1,), (1,)), ((), ())),
                             preferred_element_type=F32)
        p = jnp.exp(s_.astype(BF16))
        l = jnp.sum(p, axis=1, keepdims=True, dtype=F32)
        o = jnp.dot(p, v_ref[:, :], preferred_element_type=F32) / l
        o_ref[:, :] = o.astype(BF16)

    spec = pl.BlockSpec((S, Dh), lambda i: (i // H_loc, i % H_loc))
    return pl.pallas_call(
        body,
        grid=(B * H_loc,),
        out_shape=jax.ShapeDtypeStruct((B * S, H_loc * Dh), BF16),
        in_specs=[spec, spec, spec],
        out_specs=spec,
    )(q, k, v)


def _ring_all_reduce(my, partial_ref, acc, comm, send_sems, recv_sems,
                     ep=None):
    R = acc.shape[0]
    H = R // 2
    Ch = H // N_DEV
    right = jnp.mod(my + 1, N_DEV)
    left = jnp.mod(my + 3, N_DEV)

    def copy(src, s_off, dst, d_off, sem, tgt):
        return pltpu.make_async_remote_copy(
            src_ref=src.at[pl.ds(s_off, Ch), :],
            dst_ref=dst if d_off is None else dst.at[pl.ds(d_off, Ch), :],
            send_sem=send_sems.at[sem],
            recv_sem=recv_sems.at[sem],
            device_id=(tgt,),
            device_id_type=pl.DeviceIdType.MESH,
        )

    for s in range(N_DEV - 1):
        sc0 = jnp.mod(my - s + 8, N_DEV)
        rc0 = jnp.mod(my - s - 1 + 8, N_DEV)
        sc1 = jnp.mod(my + s, N_DEV)
        rc1 = jnp.mod(my + s + 1, N_DEV)
        src = partial_ref if s == 0 else acc
        r0 = copy(src, sc0 * Ch, comm.at[s], None, s, right)
        r1 = copy(src, H + sc1 * Ch, comm.at[3 + s], None, 3 + s, left)
        r0.start()
        r1.start()
        r0.wait()
        r1.wait()
        acc[pl.ds(rc0 * Ch, Ch), :] = (
            partial_ref[pl.ds(rc0 * Ch, Ch), :] + comm[s])
        acc[pl.ds(H + rc1 * Ch, Ch), :] = (
            partial_ref[pl.ds(H + rc1 * Ch, Ch), :] + comm[3 + s])
    for t in range(N_DEV - 1):
        sc0 = jnp.mod(my + 1 - t + 8, N_DEV)
        sc1 = jnp.mod(my - 1 + t + 8, N_DEV)
        r0 = copy(acc, sc0 * Ch, acc, sc0 * Ch, 6 + t, right)
        r1 = copy(acc, H + sc1 * Ch, acc, H + sc1 * Ch, 9 + t, left)
        r0.start()
        r1.start()
        if ep is not None:
            ep(sc0, sc1)
        r0.wait()
        r1.wait()
    if ep is not None:
        ep(jnp.mod(my - 2 + 8, N_DEV), jnp.mod(my + 2, N_DEV))


def _barriers(bar_sems):
    my = lax.axis_index("i")
    right = jnp.mod(my + 1, N_DEV)
    left = jnp.mod(my + 3, N_DEV)

    def entry():
        bsem = pltpu.get_barrier_semaphore()
        for nbr in (left, right):
            pl.semaphore_signal(bsem, inc=1, device_id=(nbr,),
                                device_id_type=pl.DeviceIdType.MESH)
        pl.semaphore_wait(bsem, 2)

    def exit_():
        for nbr in (left, right):
            pl.semaphore_signal(bar_sems.at[0], inc=1, device_id=(nbr,),
                                device_id_type=pl.DeviceIdType.MESH)
        pl.semaphore_wait(bar_sems.at[0], 2)

    return my, entry, exit_


def _ar_scratch(R, D):
    Ch = R // 2 // N_DEV
    return [
        pltpu.VMEM((R, D), BF16),
        pltpu.VMEM((6, Ch, D), BF16),
        pltpu.SemaphoreType.DMA((12,)),
        pltpu.SemaphoreType.DMA((12,)),
        pltpu.SemaphoreType.REGULAR((1,)),
    ]


def _ar1_ffn_ar2(p1, x, mod, W_ff1, W_ff2, B, S, D):
    R = B * S
    H = R // 2
    Ch = H // N_DEV

    def body(p1_ref, x_ref, mod_ref, wf1_ref, wf2_ref, out_ref,
             acc1, comm, p2, acc2, wf1b, wf2b,
             send_sems, recv_sems, bar_sems):
        my, entry, exit_ = _barriers(bar_sems)
        right = jnp.mod(my + 1, N_DEV)
        left = jnp.mod(my + 3, N_DEV)
        entry()

        def copy(src, s_off, dst, d_off, sem, tgt):
            return pltpu.make_async_remote_copy(
                src_ref=src.at[pl.ds(s_off, Ch), :],
                dst_ref=dst if d_off is None else dst.at[pl.ds(d_off, Ch), :],
                send_sem=send_sems.at[sem],
                recv_sem=recv_sems.at[sem],
                device_id=(tgt,),
                device_id_type=pl.DeviceIdType.MESH,
            )

        def md(c):
            return jnp.mod(c + 8, N_DEV)

        for s in range(N_DEV - 1):
            sc0, rc0 = md(my - s), md(my - s - 1)
            sc1, rc1 = md(my + s), md(my + s + 1)
            src = p1_ref if s == 0 else acc1
            r0 = copy(src, sc0 * Ch, comm.at[s], None, s, right)
            r1 = copy(src, H + sc1 * Ch, comm.at[3 + s], None, 3 + s, left)
            r0.start()
            r1.start()
            if s == 0:
                wf1b[:, :] = wf1_ref[:, :].astype(BF16)
                wf2b[:, :] = wf2_ref[:, :].astype(BF16)
            r0.wait()
            r1.wait()
            acc1[pl.ds(rc0 * Ch, Ch), :] = (
                p1_ref[pl.ds(rc0 * Ch, Ch), :] + comm[s])
            acc1[pl.ds(H + rc1 * Ch, Ch), :] = (
                p1_ref[pl.ds(H + rc1 * Ch, Ch), :] + comm[3 + s])

        def proc(b, ck):
            rb = pl.ds(ck * Ch, Ch)
            r = pl.ds(b * S + ck * Ch, Ch)
            x1v = (x_ref[b, rb, :]
                   + (mod_ref[b, 2 * D:3 * D][None, :]).astype(BF16)
                   * acc1[r, :])
            m = jnp.mean(x1v, axis=1, keepdims=True, dtype=F32)
            c = x1v - m.astype(BF16)
            var = jnp.mean((c * c).astype(F32), axis=1, keepdims=True)
            xln2 = (c * lax.rsqrt(var + EPS).astype(BF16)
                    * (1.0 + mod_ref[b, 3 * D:4 * D][None, :]).astype(BF16)
                    + (mod_ref[b, 4 * D:5 * D][None, :]).astype(BF16))
            h = jnp.dot(xln2, wf1b[:, :], preferred_element_type=F32)
            h = (h / (1.0 + jnp.exp(-h))).astype(BF16)
            p2[r, :] = jnp.dot(h, wf2b[:, :],
                               preferred_element_type=F32).astype(BF16)

        def rs2_hop(s, src):
            r0 = copy(src, md(my - s) * Ch, comm.at[s], None, 12 + s, right)
            r1 = copy(src, H + md(my + s) * Ch, comm.at[3 + s], None,
                      15 + s, left)
            r0.start()
            r1.start()

        def rs2_wait_acc(s, src):
            r0 = copy(src, md(my - s) * Ch, comm.at[s], None, 12 + s, right)
            r1 = copy(src, H + md(my + s) * Ch, comm.at[3 + s], None,
                      15 + s, left)
            r0.wait()
            r1.wait()
            rc0, rc1 = md(my - s - 1), md(my + s + 1)
            acc2[pl.ds(rc0 * Ch, Ch), :] = (
                p2[pl.ds(rc0 * Ch, Ch), :] + comm[s])
            acc2[pl.ds(H + rc1 * Ch, Ch), :] = (
                p2[pl.ds(H + rc1 * Ch, Ch), :] + comm[3 + s])

        def stage1(t, _):
            c0 = md(my + 1 - t)
            c1 = md(my - 1 + t)

            @pl.when(t < 3)
            def _():
                copy(acc1, c0 * Ch, acc1, c0 * Ch, 6 + t, right).start()
                copy(acc1, H + c1 * Ch, acc1, H + c1 * Ch, 9 + t,
                     left).start()

            proc(0, c0)
            proc(1, c1)

            @pl.when(t < 3)
            def _():
                copy(acc1, c0 * Ch, acc1, c0 * Ch, 6 + t, right).wait()
                copy(acc1, H + c1 * Ch, acc1, H + c1 * Ch, 9 + t,
                     left).wait()

            @pl.when(t == 1)
            def _():
                rs2_hop(0, p2)

            @pl.when(t == 2)
            def _():
                rs2_wait_acc(0, p2)
                rs2_hop(1, acc2)

            @pl.when(t == 3)
            def _():
                rs2_wait_acc(1, acc2)
                rs2_hop(2, acc2)
                rs2_wait_acc(2, acc2)

            return 0

        lax.fori_loop(0, N_DEV, stage1, 0)

        def stage2(t, _):
            c0 = md(my + 1 - t)
            c1 = md(my - 1 + t)

            @pl.when(t < 3)
            def _():
                copy(acc2, c0 * Ch, acc2, c0 * Ch, 18 + t, right).start()
                copy(acc2, H + c1 * Ch, acc2, H + c1 * Ch, 21 + t,
                     left).start()

            for b, ck in ((0, c0), (1, c1)):
                rb = pl.ds(ck * Ch, Ch)
                r = pl.ds(b * S + ck * Ch, Ch)
                out_ref[b, rb, :] = (
                    x_ref[b, rb, :].astype(F32)
                    + mod_ref[b, 2 * D:3 * D][None, :] * acc1[r, :].astype(F32)
                    + mod_ref[b, 5 * D:6 * D][None, :] * acc2[r, :].astype(F32))

            @pl.when(t < 3)
            def _():
                copy(acc2, c0 * Ch, acc2, c0 * Ch, 18 + t, right).wait()
                copy(acc2, H + c1 * Ch, acc2, H + c1 * Ch, 21 + t,
                     left).wait()

            return 0

        lax.fori_loop(0, N_DEV, stage2, 0)
        exit_()

    return pl.pallas_call(
        body,
        out_shape=jax.ShapeDtypeStruct((B, S, D), F32),
        in_specs=_vmem(5),
        out_specs=_vmem()[0],
        scratch_shapes=[
            pltpu.VMEM((R, D), BF16),
            pltpu.VMEM((6, Ch, D), BF16),
            pltpu.VMEM((R, D), BF16),
            pltpu.VMEM((R, D), BF16),
            pltpu.VMEM((D, D), BF16),
            pltpu.VMEM((D, D), BF16),
            pltpu.SemaphoreType.DMA((24,)),
            pltpu.SemaphoreType.DMA((24,)),
            pltpu.SemaphoreType.REGULAR((1,)),
        ],
        compiler_params=pltpu.CompilerParams(
            collective_id=1, vmem_limit_bytes=40 * 1024 * 1024),
    )(p1, x, mod, W_ff1, W_ff2)


def _ar1_residual_ln(p1, x, mod, B, S, D):

    Ch = S // N_DEV

    def body(p1_ref, x_ref, mod_ref, x1_ref, xln2_ref,
             acc, comm, send_sems, recv_sems, bar_sems):
        my, entry, exit_ = _barriers(bar_sems)
        entry()

        def ep(ck_cw, ck_ccw):
            for b, ck in ((0, ck_cw), (1, ck_ccw)):
                rb = pl.ds(ck * Ch, Ch)
                r = pl.ds(b * S + ck * Ch, Ch)
                x1 = (x_ref[b, rb, :]
                      + mod_ref[b, 2 * D:3 * D][None, :]
                      * acc[r, :].astype(F32))
                x1_ref[r, :] = x1
                xln2_ref[r, :] = (
                    _layernorm(x1) * (1.0 + mod_ref[b, 3 * D:4 * D][None, :])
                    + mod_ref[b, 4 * D:5 * D][None, :]).astype(BF16)

        _ring_all_reduce(my, p1_ref, acc, comm, send_sems, recv_sems, ep=ep)
        exit_()

    return pl.pallas_call(
        body,
        out_shape=[
            jax.ShapeDtypeStruct((B * S, D), F32),
            jax.ShapeDtypeStruct((B * S, D), BF16),
        ],
        in_specs=_vmem(3),
        out_specs=_vmem(2),
        scratch_shapes=_ar_scratch(B * S, D),
        compiler_params=pltpu.CompilerParams(
            collective_id=1, vmem_limit_bytes=40 * 1024 * 1024),
    )(p1, x, mod)


def _ar2_residual(p2, x1, mod, B, S, D):

    Ch = S // N_DEV

    def body(p2_ref, x1_ref, mod_ref, out_ref,
             acc, comm, send_sems, recv_sems, bar_sems):
        my, entry, exit_ = _barriers(bar_sems)
        entry()

        def ep(ck_cw, ck_ccw):
            for b, ck in ((0, ck_cw), (1, ck_ccw)):
                rb = pl.ds(ck * Ch, Ch)
                r = pl.ds(b * S + ck * Ch, Ch)
                out_ref[b, rb, :] = (x1_ref[r, :]
                                     + mod_ref[b, 5 * D:6 * D][None, :]
                                     * acc[r, :].astype(F32))

        _ring_all_reduce(my, p2_ref, acc, comm, send_sems, recv_sems, ep=ep)
        exit_()

    return pl.pallas_call(
        body,
        out_shape=jax.ShapeDtypeStruct((B, S, D), F32),
        in_specs=_vmem(3),
        out_specs=_vmem()[0],
        scratch_shapes=_ar_scratch(B * S, D),
        compiler_params=pltpu.CompilerParams(
            collective_id=2, vmem_limit_bytes=40 * 1024 * 1024),
    )(p2, x1, mod)


def _ffn(xln2, W_ff1, W_ff2):

    def body(a_ref, w1_ref, w2_ref, o_ref):
        h = jnp.dot(a_ref[:, :], w1_ref[:, :].astype(BF16),
                    preferred_element_type=F32)
        h = (h / (1.0 + jnp.exp(-h))).astype(BF16)
        o_ref[:, :] = jnp.dot(h, w2_ref[:, :].astype(BF16),
                              preferred_element_type=F32).astype(BF16)

    return pl.pallas_call(
        body,
        out_shape=jax.ShapeDtypeStruct((xln2.shape[0], W_ff2.shape[1]), BF16),
        in_specs=_vmem(3),
        out_specs=_vmem()[0],
    )(xln2, W_ff1, W_ff2)


def kernel(x, Wq, Wk, Wv, Wo, t_emb, W_mod, W_ff1, W_ff2):
    B, S, D = x.shape
    Dh = 128
    H_loc = Wq.shape[1] // Dh

    mod, xm, xbf = _mod_ln(x, t_emb, W_mod, B, S, D)
    q, k, v = _qkv(xm, Wq, Wk, Wv)
    ao = _attention(q, k, v, B, S, H_loc, Dh)
    p1 = _matmul(ao, Wo)
    return _ar1_ffn_ar2(p1, xbf, mod, W_ff1, W_ff2, B, S, D)


# device time: 161968 ns/iter; 1.3083x vs baseline; 1.0244x over previous
import jax
import jax.numpy as jnp
from jax import lax
from jax.experimental import pallas as pl
from jax.experimental.pallas import tpu as pltpu

N_DEV = 4
EPS = 1e-5
SCALE = 0.08838834764831843
BF16 = jnp.bfloat16
F32 = jnp.float32


def _vmem(n=1):
    return [pl.BlockSpec(memory_space=pltpu.VMEM)] * n


def _matmul(a, w, out_dtype=BF16):
    M = a.shape[0]
    N = w.shape[1]

    def body(a_ref, w_ref, o_ref):
        o = jnp.dot(a_ref[:, :].astype(BF16), w_ref[:, :].astype(BF16),
                    preferred_element_type=F32)
        o_ref[:, :] = o.astype(out_dtype)

    return pl.pallas_call(
        body,
        out_shape=jax.ShapeDtypeStruct((M, N), out_dtype),
        in_specs=_vmem(2),
        out_specs=_vmem()[0],
    )(a, w)


def _layernorm(h):
    m = jnp.mean(h, axis=1, keepdims=True)
    c = h - m
    var = jnp.mean(c * c, axis=1, keepdims=True)
    return c * lax.rsqrt(var + EPS)


def _mod_ln(x, t_emb, W_mod, B, S, D):

    def body(x_ref, temb_ref, wmod_ref, mod_ref, xm_ref, xbf_ref):
        mod = jnp.dot(temb_ref[:, :], wmod_ref[:, :],
                      preferred_element_type=F32)
        mod_ref[:, :] = mod
        for b in range(B):
            xv = x_ref[b, :, :]
            xbf_ref[b, :, :] = xv.astype(BF16)
            xln = _layernorm(xv)
            xm_ref[b * S:(b + 1) * S, :] = (
                xln * (1.0 + mod[b, 0:D][None, :])
                + mod[b, D:2 * D][None, :]).astype(BF16)

    return pl.pallas_call(
        body,
        out_shape=[
            jax.ShapeDtypeStruct((B, 6 * D), F32),
            jax.ShapeDtypeStruct((B * S, D), BF16),
            jax.ShapeDtypeStruct((B, S, D), BF16),
        ],
        in_specs=_vmem(3),
        out_specs=_vmem(3),
    )(x, t_emb, W_mod)


def _qkv(xm, Wq, Wk, Wv, Wo):
    M, D = xm.shape

    def body(a_ref, wq_ref, wk_ref, wv_ref, wo_ref,
             q_ref, k_ref, v_ref, wob_ref):
        a = a_ref[:, :]
        q_ref[:, :] = (jnp.dot(a, wq_ref[:, :].astype(BF16),
                               preferred_element_type=F32) * SCALE).astype(BF16)
        k_ref[:, :] = jnp.dot(a, wk_ref[:, :].astype(BF16),
                              preferred_element_type=F32).astype(BF16)
        v_ref[:, :] = jnp.dot(a, wv_ref[:, :].astype(BF16),
                              preferred_element_type=F32).astype(BF16)
        wob_ref[:, :] = wo_ref[:, :].astype(BF16)

    sh = jax.ShapeDtypeStruct((M, Wq.shape[1]), BF16)
    return pl.pallas_call(
        body,
        out_shape=[sh, sh, sh,
                   jax.ShapeDtypeStruct(Wo.shape, BF16)],
        in_specs=_vmem(5),
        out_specs=_vmem(4),
    )(xm, Wq, Wk, Wv, Wo)


def _attention(q, k, v, B, S, H_loc, Dh):

    def body(q_ref, k_ref, v_ref, o_ref):
        s_ = lax.dot_general(q_ref[:, :], k_ref[:, :],
                             (((1,), (1,)), ((), ())),
                             preferred_element_type=F32)
        p = jnp.exp(s_.astype(BF16))
        l = jnp.sum(p, axis=1, keepdims=True, dtype=F32)
        o = jnp.dot(p, v_ref[:, :], preferred_element_type=F32) / l
        o_ref[:, :] = o.astype(BF16)

    spec = pl.BlockSpec((S, Dh), lambda i: (i // H_loc, i % H_loc))
    return pl.pallas_call(
        body,
        grid=(B * H_loc,),
        out_shape=jax.ShapeDtypeStruct((B * S, H_loc * Dh), BF16),
        in_specs=[spec, spec, spec],
        out_specs=spec,
    )(q, k, v)


def _ring_all_reduce(my, partial_ref, acc, comm, send_sems, recv_sems,
                     ep=None):
    R = acc.shape[0]
    H = R // 2
    Ch = H // N_DEV
    right = jnp.mod(my + 1, N_DEV)
    left = jnp.mod(my + 3, N_DEV)

    def copy(src, s_off, dst, d_off, sem, tgt):
        return pltpu.make_async_remote_copy(
            src_ref=src.at[pl.ds(s_off, Ch), :],
            dst_ref=dst if d_off is None else dst.at[pl.ds(d_off, Ch), :],
            send_sem=send_sems.at[sem],
            recv_sem=recv_sems.at[sem],
            device_id=(tgt,),
            device_id_type=pl.DeviceIdType.MESH,
        )

    for s in range(N_DEV - 1):
        sc0 = jnp.mod(my - s + 8, N_DEV)
        rc0 = jnp.mod(my - s - 1 + 8, N_DEV)
        sc1 = jnp.mod(my + s, N_DEV)
        rc1 = jnp.mod(my + s + 1, N_DEV)
        src = partial_ref if s == 0 else acc
        r0 = copy(src, sc0 * Ch, comm.at[s], None, s, right)
        r1 = copy(src, H + sc1 * Ch, comm.at[3 + s], None, 3 + s, left)
        r0.start()
        r1.start()
        r0.wait()
        r1.wait()
        acc[pl.ds(rc0 * Ch, Ch), :] = (
            partial_ref[pl.ds(rc0 * Ch, Ch), :] + comm[s])
        acc[pl.ds(H + rc1 * Ch, Ch), :] = (
            partial_ref[pl.ds(H + rc1 * Ch, Ch), :] + comm[3 + s])
    for t in range(N_DEV - 1):
        sc0 = jnp.mod(my + 1 - t + 8, N_DEV)
        sc1 = jnp.mod(my - 1 + t + 8, N_DEV)
        r0 = copy(acc, sc0 * Ch, acc, sc0 * Ch, 6 + t, right)
        r1 = copy(acc, H + sc1 * Ch, acc, H + sc1 * Ch, 9 + t, left)
        r0.start()
        r1.start()
        if ep is not None:
            ep(sc0, sc1)
        r0.wait()
        r1.wait()
    if ep is not None:
        ep(jnp.mod(my - 2 + 8, N_DEV), jnp.mod(my + 2, N_DEV))


def _barriers(bar_sems):
    my = lax.axis_index("i")
    right = jnp.mod(my + 1, N_DEV)
    left = jnp.mod(my + 3, N_DEV)

    def entry():
        bsem = pltpu.get_barrier_semaphore()
        for nbr in (left, right):
            pl.semaphore_signal(bsem, inc=1, device_id=(nbr,),
                                device_id_type=pl.DeviceIdType.MESH)
        pl.semaphore_wait(bsem, 2)

    def exit_():
        for nbr in (left, right):
            pl.semaphore_signal(bar_sems.at[0], inc=1, device_id=(nbr,),
                                device_id_type=pl.DeviceIdType.MESH)
        pl.semaphore_wait(bar_sems.at[0], 2)

    return my, entry, exit_


def _ar_scratch(R, D):
    Ch = R // 2 // N_DEV
    return [
        pltpu.VMEM((R, D), BF16),
        pltpu.VMEM((6, Ch, D), BF16),
        pltpu.SemaphoreType.DMA((12,)),
        pltpu.SemaphoreType.DMA((12,)),
        pltpu.SemaphoreType.REGULAR((1,)),
    ]


def _ar1_ffn_ar2(ao, x, mod, wo_bf, W_ff1, W_ff2, B, S, D):
    R = B * S
    H = R // 2
    Ch = H // N_DEV

    def body(ao_ref, x_ref, mod_ref, wob_ref, wf1_ref, wf2_ref, out_ref,
             acc1, comm, p2, acc2, wf1b, wf2b,
             send_sems, recv_sems, bar_sems):
        my, entry, exit_ = _barriers(bar_sems)
        right = jnp.mod(my + 1, N_DEV)
        left = jnp.mod(my + 3, N_DEV)
        entry()

        def copy(src, s_off, dst, d_off, sem, tgt):
            return pltpu.make_async_remote_copy(
                src_ref=src.at[pl.ds(s_off, Ch), :],
                dst_ref=dst if d_off is None else dst.at[pl.ds(d_off, Ch), :],
                send_sem=send_sems.at[sem],
                recv_sem=recv_sems.at[sem],
                device_id=(tgt,),
                device_id_type=pl.DeviceIdType.MESH,
            )

        def md(c):
            return jnp.mod(c + 8, N_DEV)

        def rs_hop(s, src, base):
            copy(src, md(my - s) * Ch, comm.at[s], None, base + s,
                 right).start()
            copy(src, H + md(my + s) * Ch, comm.at[3 + s], None,
                 base + 3 + s, left).start()

        def rs_wait_acc(s, src, base, part, accbuf):
            copy(src, md(my - s) * Ch, comm.at[s], None, base + s,
                 right).wait()
            copy(src, H + md(my + s) * Ch, comm.at[3 + s], None,
                 base + 3 + s, left).wait()
            rc0, rc1 = md(my - s - 1), md(my + s + 1)
            accbuf[pl.ds(rc0 * Ch, Ch), :] = (
                part[pl.ds(rc0 * Ch, Ch), :] + comm[s])
            accbuf[pl.ds(H + rc1 * Ch, Ch), :] = (
                part[pl.ds(H + rc1 * Ch, Ch), :] + comm[3 + s])

        def proc_p1(b, ck):
            r = pl.ds(b * S + ck * Ch, Ch)
            acc1[r, :] = jnp.dot(ao_ref[r, :], wob_ref[:, :],
                                 preferred_element_type=F32).astype(BF16)

        def stage0(s, _):
            c0, c1 = md(my - s), md(my + s)
            proc_p1(0, c0)
            proc_p1(1, c1)

            @pl.when(s == 1)
            def _():
                rs_wait_acc(0, acc1, 0, acc1, acc1)
                rs_hop(1, acc1, 0)

            @pl.when(s == 2)
            def _():
                rs_wait_acc(1, acc1, 0, acc1, acc1)
                rs_hop(2, acc1, 0)

            @pl.when(s == 3)
            def _():
                rs_wait_acc(2, acc1, 0, acc1, acc1)

            @pl.when(s == 0)
            def _():
                rs_hop(0, acc1, 0)
                wf1b[:, :] = wf1_ref[:, :].astype(BF16)
                wf2b[:, :] = wf2_ref[:, :].astype(BF16)

            return 0

        lax.fori_loop(0, N_DEV, stage0, 0)

        def proc(b, ck):
            rb = pl.ds(ck * Ch, Ch)
            r = pl.ds(b * S + ck * Ch, Ch)
            x1v = (x_ref[b, rb, :]
                   + (mod_ref[b, 2 * D:3 * D][None, :]).astype(BF16)
                   * acc1[r, :])
            m = jnp.mean(x1v, axis=1, keepdims=True, dtype=F32)
            c = x1v - m.astype(BF16)
            var = jnp.mean((c * c).astype(F32), axis=1, keepdims=True)
            xln2 = (c * lax.rsqrt(var + EPS).astype(BF16)
                    * (1.0 + mod_ref[b, 3 * D:4 * D][None, :]).astype(BF16)
                    + (mod_ref[b, 4 * D:5 * D][None, :]).astype(BF16))
            h = jnp.dot(xln2, wf1b[:, :], preferred_element_type=F32)
            h = (h / (1.0 + jnp.exp(-h))).astype(BF16)
            p2[r, :] = jnp.dot(h, wf2b[:, :],
                               preferred_element_type=F32).astype(BF16)

        def rs2_hop(s, src):
            r0 = copy(src, md(my - s) * Ch, comm.at[s], None, 12 + s, right)
            r1 = copy(src, H + md(my + s) * Ch, comm.at[3 + s], None,
                      15 + s, left)
            r0.start()
            r1.start()

        def rs2_wait_acc(s, src):
            r0 = copy(src, md(my - s) * Ch, comm.at[s], None, 12 + s, right)
            r1 = copy(src, H + md(my + s) * Ch, comm.at[3 + s], None,
                      15 + s, left)
            r0.wait()
            r1.wait()
            rc0, rc1 = md(my - s - 1), md(my + s + 1)
            acc2[pl.ds(rc0 * Ch, Ch), :] = (
                p2[pl.ds(rc0 * Ch, Ch), :] + comm[s])
            acc2[pl.ds(H + rc1 * Ch, Ch), :] = (
                p2[pl.ds(H + rc1 * Ch, Ch), :] + comm[3 + s])

        def stage1(t, _):
            c0 = md(my + 1 - t)
            c1 = md(my - 1 + t)

            @pl.when(t < 3)
            def _():
                copy(acc1, c0 * Ch, acc1, c0 * Ch, 6 + t, right).start()
                copy(acc1, H + c1 * Ch, acc1, H + c1 * Ch, 9 + t,
                     left).start()

            proc(0, c0)
            proc(1, c1)

            @pl.when(t < 3)
            def _():
                copy(acc1, c0 * Ch, acc1, c0 * Ch, 6 + t, right).wait()
                copy(acc1, H + c1 * Ch, acc1, H + c1 * Ch, 9 + t,
                     left).wait()

            @pl.when(t == 1)
            def _():
                rs2_hop(0, p2)

            @pl.when(t == 2)
            def _():
                rs2_wait_acc(0, p2)
                rs2_hop(1, acc2)

            @pl.when(t == 3)
            def _():
                rs2_wait_acc(1, acc2)
                rs2_hop(2, acc2)
                rs2_wait_acc(2, acc2)

            return 0

        lax.fori_loop(0, N_DEV, stage1, 0)

        def stage2(t, _):
            c0 = md(my + 1 - t)
            c1 = md(my - 1 + t)

            @pl.when(t < 3)
            def _():
                copy(acc2, c0 * Ch, acc2, c0 * Ch, 18 + t, right).start()
                copy(acc2, H + c1 * Ch, acc2, H + c1 * Ch, 21 + t,
                     left).start()

            for b, ck in ((0, c0), (1, c1)):
                rb = pl.ds(ck * Ch, Ch)
                r = pl.ds(b * S + ck * Ch, Ch)
                out_ref[b, rb, :] = (
                    x_ref[b, rb, :].astype(F32)
                    + mod_ref[b, 2 * D:3 * D][None, :] * acc1[r, :].astype(F32)
                    + mod_ref[b, 5 * D:6 * D][None, :] * acc2[r, :].astype(F32))

            @pl.when(t < 3)
            def _():
                copy(acc2, c0 * Ch, acc2, c0 * Ch, 18 + t, right).wait()
                copy(acc2, H + c1 * Ch, acc2, H + c1 * Ch, 21 + t,
                     left).wait()

            return 0

        lax.fori_loop(0, N_DEV, stage2, 0)
        exit_()

    return pl.pallas_call(
        body,
        out_shape=jax.ShapeDtypeStruct((B, S, D), F32),
        in_specs=_vmem(6),
        out_specs=_vmem()[0],
        scratch_shapes=[
            pltpu.VMEM((R, D), BF16),
            pltpu.VMEM((6, Ch, D), BF16),
            pltpu.VMEM((R, D), BF16),
            pltpu.VMEM((R, D), BF16),
            pltpu.VMEM((D, D), BF16),
            pltpu.VMEM((D, D), BF16),
            pltpu.SemaphoreType.DMA((24,)),
            pltpu.SemaphoreType.DMA((24,)),
            pltpu.SemaphoreType.REGULAR((1,)),
        ],
        compiler_params=pltpu.CompilerParams(
            collective_id=1, vmem_limit_bytes=40 * 1024 * 1024),
    )(ao, x, mod, wo_bf, W_ff1, W_ff2)


def _ar1_residual_ln(p1, x, mod, B, S, D):

    Ch = S // N_DEV

    def body(p1_ref, x_ref, mod_ref, x1_ref, xln2_ref,
             acc, comm, send_sems, recv_sems, bar_sems):
        my, entry, exit_ = _barriers(bar_sems)
        entry()

        def ep(ck_cw, ck_ccw):
            for b, ck in ((0, ck_cw), (1, ck_ccw)):
                rb = pl.ds(ck * Ch, Ch)
                r = pl.ds(b * S + ck * Ch, Ch)
                x1 = (x_ref[b, rb, :]
                      + mod_ref[b, 2 * D:3 * D][None, :]
                      * acc[r, :].astype(F32))
                x1_ref[r, :] = x1
                xln2_ref[r, :] = (
                    _layernorm(x1) * (1.0 + mod_ref[b, 3 * D:4 * D][None, :])
                    + mod_ref[b, 4 * D:5 * D][None, :]).astype(BF16)

        _ring_all_reduce(my, p1_ref, acc, comm, send_sems, recv_sems, ep=ep)
        exit_()

    return pl.pallas_call(
        body,
        out_shape=[
            jax.ShapeDtypeStruct((B * S, D), F32),
            jax.ShapeDtypeStruct((B * S, D), BF16),
        ],
        in_specs=_vmem(3),
        out_specs=_vmem(2),
        scratch_shapes=_ar_scratch(B * S, D),
        compiler_params=pltpu.CompilerParams(
            collective_id=1, vmem_limit_bytes=40 * 1024 * 1024),
    )(p1, x, mod)


def _ar2_residual(p2, x1, mod, B, S, D):

    Ch = S // N_DEV

    def body(p2_ref, x1_ref, mod_ref, out_ref,
             acc, comm, send_sems, recv_sems, bar_sems):
        my, entry, exit_ = _barriers(bar_sems)
        entry()

        def ep(ck_cw, ck_ccw):
            for b, ck in ((0, ck_cw), (1, ck_ccw)):
                rb = pl.ds(ck * Ch, Ch)
                r = pl.ds(b * S + ck * Ch, Ch)
                out_ref[b, rb, :] = (x1_ref[r, :]
                                     + mod_ref[b, 5 * D:6 * D][None, :]
                                     * acc[r, :].astype(F32))

        _ring_all_reduce(my, p2_ref, acc, comm, send_sems, recv_sems, ep=ep)
        exit_()

    return pl.pallas_call(
        body,
        out_shape=jax.ShapeDtypeStruct((B, S, D), F32),
        in_specs=_vmem(3),
        out_specs=_vmem()[0],
        scratch_shapes=_ar_scratch(B * S, D),
        compiler_params=pltpu.CompilerParams(
            collective_id=2, vmem_limit_bytes=40 * 1024 * 1024),
    )(p2, x1, mod)


def _ffn(xln2, W_ff1, W_ff2):

    def body(a_ref, w1_ref, w2_ref, o_ref):
        h = jnp.dot(a_ref[:, :], w1_ref[:, :].astype(BF16),
                    preferred_element_type=F32)
        h = (h / (1.0 + jnp.exp(-h))).astype(BF16)
        o_ref[:, :] = jnp.dot(h, w2_ref[:, :].astype(BF16),
                              preferred_element_type=F32).astype(BF16)

    return pl.pallas_call(
        body,
        out_shape=jax.ShapeDtypeStruct((xln2.shape[0], W_ff2.shape[1]), BF16),
        in_specs=_vmem(3),
        out_specs=_vmem()[0],
    )(xln2, W_ff1, W_ff2)


def kernel(x, Wq, Wk, Wv, Wo, t_emb, W_mod, W_ff1, W_ff2):
    B, S, D = x.shape
    Dh = 128
    H_loc = Wq.shape[1] // Dh

    mod, xm, xbf = _mod_ln(x, t_emb, W_mod, B, S, D)
    q, k, v, wo_bf = _qkv(xm, Wq, Wk, Wv, Wo)
    ao = _attention(q, k, v, B, S, H_loc, Dh)
    return _ar1_ffn_ar2(ao, xbf, mod, wo_bf, W_ff1, W_ff2, B, S, D)


# device time: 153591 ns/iter; 1.3797x vs baseline; 1.0545x over previous
import jax
import jax.numpy as jnp
from jax import lax
from jax.experimental import pallas as pl
from jax.experimental.pallas import tpu as pltpu

N_DEV = 4
EPS = 1e-5
SCALE = 0.08838834764831843
BF16 = jnp.bfloat16
F32 = jnp.float32


def _vmem(n=1):
    return [pl.BlockSpec(memory_space=pltpu.VMEM)] * n


def _layernorm(h):
    m = jnp.mean(h, axis=1, keepdims=True)
    c = h - m
    var = jnp.mean(c * c, axis=1, keepdims=True)
    return c * lax.rsqrt(var + EPS)


def _mod_ln(x, t_emb, W_mod, B, S, D):

    def body(x_ref, temb_ref, wmod_ref, mod_ref, xm_ref, xbf_ref):
        mod = jnp.dot(temb_ref[:, :], wmod_ref[:, :],
                      preferred_element_type=F32)
        mod_ref[:, :] = mod
        for b in range(B):
            xv = x_ref[b, :, :]
            xbf_ref[b, :, :] = xv.astype(BF16)
            xln = _layernorm(xv)
            xm_ref[b * S:(b + 1) * S, :] = (
                xln * (1.0 + mod[b, 0:D][None, :])
                + mod[b, D:2 * D][None, :]).astype(BF16)

    return pl.pallas_call(
        body,
        out_shape=[
            jax.ShapeDtypeStruct((B, 6 * D), F32),
            jax.ShapeDtypeStruct((B * S, D), BF16),
            jax.ShapeDtypeStruct((B, S, D), BF16),
        ],
        in_specs=_vmem(3),
        out_specs=_vmem(3),
    )(x, t_emb, W_mod)


def _qkv(xm, Wq, Wk, Wv, Wo):
    M, D = xm.shape

    def body(a_ref, wq_ref, wk_ref, wv_ref, wo_ref,
             q_ref, k_ref, v_ref, wob_ref):
        a = a_ref[:, :]
        q_ref[:, :] = (jnp.dot(a, wq_ref[:, :].astype(BF16),
                               preferred_element_type=F32) * SCALE).astype(BF16)
        k_ref[:, :] = jnp.dot(a, wk_ref[:, :].astype(BF16),
                              preferred_element_type=F32).astype(BF16)
        v_ref[:, :] = jnp.dot(a, wv_ref[:, :].astype(BF16),
                              preferred_element_type=F32).astype(BF16)
        wob_ref[:, :] = wo_ref[:, :].astype(BF16)

    sh = jax.ShapeDtypeStruct((M, Wq.shape[1]), BF16)
    return pl.pallas_call(
        body,
        out_shape=[sh, sh, sh,
                   jax.ShapeDtypeStruct(Wo.shape, BF16)],
        in_specs=_vmem(5),
        out_specs=_vmem(4),
    )(xm, Wq, Wk, Wv, Wo)


def _attention(q, k, v, B, S, H_loc, Dh):

    def body(q_ref, k_ref, v_ref, o_ref):
        s_ = lax.dot_general(q_ref[:, :], k_ref[:, :],
                             (((1,), (1,)), ((), ())),
                             preferred_element_type=F32)
        p = jnp.exp(s_.astype(BF16))
        l = jnp.sum(p, axis=1, keepdims=True, dtype=F32)
        o = jnp.dot(p, v_ref[:, :], preferred_element_type=F32) / l
        o_ref[:, :] = o.astype(BF16)

    spec = pl.BlockSpec((S, Dh), lambda i: (i // H_loc, i % H_loc))
    return pl.pallas_call(
        body,
        grid=(B * H_loc,),
        out_shape=jax.ShapeDtypeStruct((B * S, H_loc * Dh), BF16),
        in_specs=[spec, spec, spec],
        out_specs=spec,
    )(q, k, v)


def _barriers(bar_sems):
    my = lax.axis_index("i")
    right = jnp.mod(my + 1, N_DEV)
    left = jnp.mod(my + 3, N_DEV)

    def entry():
        bsem = pltpu.get_barrier_semaphore()
        for nbr in (left, right):
            pl.semaphore_signal(bsem, inc=1, device_id=(nbr,),
                                device_id_type=pl.DeviceIdType.MESH)
        pl.semaphore_wait(bsem, 2)

    def exit_():
        for nbr in (left, right):
            pl.semaphore_signal(bar_sems.at[0], inc=1, device_id=(nbr,),
                                device_id_type=pl.DeviceIdType.MESH)
        pl.semaphore_wait(bar_sems.at[0], 2)

    return my, entry, exit_


def _ar1_ffn_ar2(ao, x, mod, wo_bf, W_ff1, W_ff2, B, S, D):
    R = B * S
    H = R // 2
    Ch = H // N_DEV
    Cs = Ch // 2

    def body(ao_ref, x_ref, mod_ref, wob_ref, wf1_ref, wf2_ref, out_ref,
             acc1, comm1, comm2, p2, acc2, wf1b, wf2b,
             send_sems, recv_sems, bar_sems):
        my, entry, exit_ = _barriers(bar_sems)
        right = jnp.mod(my + 1, N_DEV)
        left = jnp.mod(my + 3, N_DEV)
        tgt = (right, left)
        entry()

        def md(c):
            return jnp.mod(c + 8, N_DEV)

        def rcopy(src_ref, s_off, n, dst_ref, d_off, sem, d):
            return pltpu.make_async_remote_copy(
                src_ref=src_ref.at[pl.ds(s_off, n), :],
                dst_ref=(dst_ref if d_off is None
                         else dst_ref.at[pl.ds(d_off, n), :]),
                send_sem=send_sems.at[sem],
                recv_sem=recv_sems.at[sem],
                device_id=(tgt[d],),
                device_id_type=pl.DeviceIdType.MESH,
            )

        def sub_off(ck, d, u):
            return d * H + ck * Ch + u * Cs

        def rs1_send(s, d, u):
            ck = md(my - s) if d == 0 else md(my + s)
            rcopy(acc1, sub_off(ck, d, u), Cs,
                  comm1.at[s * 4 + d * 2 + u], None,
                  s * 4 + d * 2 + u, d).start()

        def rs1_wait_acc(s, d, u):
            ck = md(my - s) if d == 0 else md(my + s)
            rcopy(acc1, sub_off(ck, d, u), Cs,
                  comm1.at[s * 4 + d * 2 + u], None,
                  s * 4 + d * 2 + u, d).wait()
            rc = md(my - s - 1) if d == 0 else md(my + s + 1)
            o = sub_off(rc, d, u)
            acc1[pl.ds(o, Cs), :] = (
                acc1[pl.ds(o, Cs), :] + comm1[s * 4 + d * 2 + u])

        def proc_p1(b, ck):
            r = pl.ds(b * S + ck * Ch, Ch)
            acc1[r, :] = jnp.dot(ao_ref[r, :], wob_ref[:, :],
                                 preferred_element_type=F32).astype(BF16)

        def stage0(s, _):
            c0, c1 = md(my - s), md(my + s)
            proc_p1(0, c0)
            proc_p1(1, c1)

            @pl.when(s == 0)
            def _():
                for u in (0, 1):
                    for d in (0, 1):
                        rs1_send(0, d, u)
                wf1b[:, :] = wf1_ref[:, :].astype(BF16)
                wf2b[:, :] = wf2_ref[:, :].astype(BF16)

            @pl.when(s == 1)
            def _():
                for u in (0, 1):
                    for d in (0, 1):
                        rs1_wait_acc(0, d, u)
                        rs1_send(1, d, u)

            @pl.when(s == 2)
            def _():
                for u in (0, 1):
                    for d in (0, 1):
                        rs1_wait_acc(1, d, u)
                        rs1_send(2, d, u)

            @pl.when(s == 3)
            def _():
                for u in (0, 1):
                    for d in (0, 1):
                        rs1_wait_acc(2, d, u)

            return 0

        lax.fori_loop(0, N_DEV, stage0, 0)

        def proc(b, ck):
            rb = pl.ds(ck * Ch, Ch)
            r = pl.ds(b * S + ck * Ch, Ch)
            x1v = (x_ref[b, rb, :]
                   + (mod_ref[b, 2 * D:3 * D][None, :]).astype(BF16)
                   * acc1[r, :])
            m = jnp.mean(x1v, axis=1, keepdims=True, dtype=F32)
            c = x1v - m.astype(BF16)
            var = jnp.mean((c * c).astype(F32), axis=1, keepdims=True)
            xln2 = (c * lax.rsqrt(var + EPS).astype(BF16)
                    * (1.0 + mod_ref[b, 3 * D:4 * D][None, :]).astype(BF16)
                    + (mod_ref[b, 4 * D:5 * D][None, :]).astype(BF16))
            h = jnp.dot(xln2, wf1b[:, :], preferred_element_type=F32)
            h = (h / (1.0 + jnp.exp(-h))).astype(BF16)
            p2[r, :] = jnp.dot(h, wf2b[:, :],
                               preferred_element_type=F32).astype(BF16)

        def rs2_hop(s, src):
            rcopy(src, md(my - s) * Ch, Ch, comm2.at[s], None,
                  24 + s, 0).start()
            rcopy(src, H + md(my + s) * Ch, Ch, comm2.at[3 + s], None,
                  27 + s, 1).start()

        def rs2_wait_acc(s, src):
            rcopy(src, md(my - s) * Ch, Ch, comm2.at[s], None,
                  24 + s, 0).wait()
            rcopy(src, H + md(my + s) * Ch, Ch, comm2.at[3 + s], None,
                  27 + s, 1).wait()
            rc0, rc1 = md(my - s - 1), md(my + s + 1)
            acc2[pl.ds(rc0 * Ch, Ch), :] = (
                p2[pl.ds(rc0 * Ch, Ch), :] + comm2[s])
            acc2[pl.ds(H + rc1 * Ch, Ch), :] = (
                p2[pl.ds(H + rc1 * Ch, Ch), :] + comm2[3 + s])

        def ag1_io(t, d, u, start):
            ck = md(my + 1 - t) if d == 0 else md(my - 1 + t)
            o = sub_off(ck, d, u)
            r = rcopy(acc1, o, Cs, acc1, o, 12 + t * 4 + d * 2 + u, d)
            r.start() if start else r.wait()

        def stage1(t, _):
            for u in (0, 1):
                @pl.when(t > 0)
                def _():
                    for d in (0, 1):
                        ag1_io(t - 1, d, u, False)

                @pl.when(t < 3)
                def _():
                    for d in (0, 1):
                        ag1_io(t, d, u, True)

            proc(0, md(my + 1 - t))
            proc(1, md(my - 1 + t))

            @pl.when(t == 1)
            def _():
                rs2_hop(0, p2)

            @pl.when(t == 2)
            def _():
                rs2_wait_acc(0, p2)
                rs2_hop(1, acc2)

            @pl.when(t == 3)
            def _():
                rs2_wait_acc(1, acc2)
                rs2_hop(2, acc2)
                rs2_wait_acc(2, acc2)

            return 0

        lax.fori_loop(0, N_DEV, stage1, 0)

        def ag2_io(t, d, u, start):
            ck = md(my + 1 - t) if d == 0 else md(my - 1 + t)
            o = sub_off(ck, d, u)
            r = rcopy(acc2, o, Cs, acc2, o, 30 + t * 4 + d * 2 + u, d)
            r.start() if start else r.wait()

        def stage2(t, _):
            for u in (0, 1):
                @pl.when(t > 0)
                def _():
                    for d in (0, 1):
                        ag2_io(t - 1, d, u, False)

                @pl.when(t < 3)
                def _():
                    for d in (0, 1):
                        ag2_io(t, d, u, True)

            for b, ck in ((0, md(my + 1 - t)), (1, md(my - 1 + t))):
                rb = pl.ds(ck * Ch, Ch)
                r = pl.ds(b * S + ck * Ch, Ch)
                out_ref[b, rb, :] = (
                    x_ref[b, rb, :].astype(F32)
                    + mod_ref[b, 2 * D:3 * D][None, :] * acc1[r, :].astype(F32)
                    + mod_ref[b, 5 * D:6 * D][None, :] * acc2[r, :].astype(F32))

            return 0

        lax.fori_loop(0, N_DEV, stage2, 0)
        exit_()

    return pl.pallas_call(
        body,
        out_shape=jax.ShapeDtypeStruct((B, S, D), F32),
        in_specs=_vmem(6),
        out_specs=_vmem()[0],
        scratch_shapes=[
            pltpu.VMEM((R, D), BF16),
            pltpu.VMEM((12, Cs, D), BF16),
            pltpu.VMEM((6, Ch, D), BF16),
            pltpu.VMEM((R, D), BF16),
            pltpu.VMEM((R, D), BF16),
            pltpu.VMEM((D, D), BF16),
            pltpu.VMEM((D, D), BF16),
            pltpu.SemaphoreType.DMA((42,)),
            pltpu.SemaphoreType.DMA((42,)),
            pltpu.SemaphoreType.REGULAR((1,)),
        ],
        compiler_params=pltpu.CompilerParams(
            collective_id=1, vmem_limit_bytes=44 * 1024 * 1024),
    )(ao, x, mod, wo_bf, W_ff1, W_ff2)


def kernel(x, Wq, Wk, Wv, Wo, t_emb, W_mod, W_ff1, W_ff2):
    B, S, D = x.shape
    Dh = 128
    H_loc = Wq.shape[1] // Dh

    mod, xm, xbf = _mod_ln(x, t_emb, W_mod, B, S, D)
    q, k, v, wo_bf = _qkv(xm, Wq, Wk, Wv, Wo)
    ao = _attention(q, k, v, B, S, H_loc, Dh)
    return _ar1_ffn_ar2(ao, xbf, mod, wo_bf, W_ff1, W_ff2, B, S, D)


# device time: 151085 ns/iter; 1.4025x vs baseline; 1.0166x over previous
import jax
import jax.numpy as jnp
from jax import lax
from jax.experimental import pallas as pl
from jax.experimental.pallas import tpu as pltpu

N_DEV = 4
EPS = 1e-5
SCALE = 0.08838834764831843
BF16 = jnp.bfloat16
F32 = jnp.float32


def _vmem(n=1):
    return [pl.BlockSpec(memory_space=pltpu.VMEM)] * n


def _layernorm(h):
    m = jnp.mean(h, axis=1, keepdims=True)
    c = h - m
    var = jnp.mean(c * c, axis=1, keepdims=True)
    return c * lax.rsqrt(var + EPS)


def _mod_ln(x, t_emb, W_mod, B, S, D):

    def body(x_ref, temb_ref, wmod_ref, mod_ref, xm_ref, xbf_ref):
        mod = jnp.dot(temb_ref[:, :], wmod_ref[:, :],
                      preferred_element_type=F32)
        mod_ref[:, :] = mod
        for b in range(B):
            xv = x_ref[b, :, :]
            xbf_ref[b, :, :] = xv.astype(BF16)
            xln = _layernorm(xv)
            xm_ref[b * S:(b + 1) * S, :] = (
                xln * (1.0 + mod[b, 0:D][None, :])
                + mod[b, D:2 * D][None, :]).astype(BF16)

    return pl.pallas_call(
        body,
        out_shape=[
            jax.ShapeDtypeStruct((B, 6 * D), F32),
            jax.ShapeDtypeStruct((B * S, D), BF16),
            jax.ShapeDtypeStruct((B, S, D), BF16),
        ],
        in_specs=_vmem(3),
        out_specs=_vmem(3),
    )(x, t_emb, W_mod)


def _qkv(xm, Wq, Wk, Wv, Wo):
    M, D = xm.shape

    def body(a_ref, wq_ref, wk_ref, wv_ref, wo_ref,
             q_ref, k_ref, v_ref, wob_ref):
        a = a_ref[:, :]
        q_ref[:, :] = (jnp.dot(a, wq_ref[:, :].astype(BF16),
                               preferred_element_type=F32) * SCALE).astype(BF16)
        k_ref[:, :] = jnp.dot(a, wk_ref[:, :].astype(BF16),
                              preferred_element_type=F32).astype(BF16)
        v_ref[:, :] = jnp.dot(a, wv_ref[:, :].astype(BF16),
                              preferred_element_type=F32).astype(BF16)
        wob_ref[:, :] = wo_ref[:, :].astype(BF16)

    blk = 256
    N = Wq.shape[1]
    sh = jax.ShapeDtypeStruct((M, N), BF16)
    wspec = pl.BlockSpec((D, blk), lambda i: (0, i))
    return pl.pallas_call(
        body,
        grid=(N // blk,),
        out_shape=[sh, sh, sh,
                   jax.ShapeDtypeStruct(Wo.shape, BF16)],
        in_specs=[pl.BlockSpec((M, D), lambda i: (0, 0)),
                  wspec, wspec, wspec, wspec],
        out_specs=[pl.BlockSpec((M, blk), lambda i: (0, i))] * 3
        + [wspec],
    )(xm, Wq, Wk, Wv, Wo)


def _attention(q, k, v, B, S, H_loc, Dh):

    def body(q_ref, k_ref, v_ref, o_ref):
        s_ = lax.dot_general(q_ref[:, :], k_ref[:, :],
                             (((1,), (1,)), ((), ())),
                             preferred_element_type=F32)
        p = jnp.exp(s_.astype(BF16))
        l = jnp.sum(p, axis=1, keepdims=True, dtype=F32)
        o = jnp.dot(p, v_ref[:, :], preferred_element_type=F32) / l
        o_ref[:, :] = o.astype(BF16)

    spec = pl.BlockSpec((S, Dh), lambda i: (i // H_loc, i % H_loc))
    return pl.pallas_call(
        body,
        grid=(B * H_loc,),
        out_shape=jax.ShapeDtypeStruct((B * S, H_loc * Dh), BF16),
        in_specs=[spec, spec, spec],
        out_specs=spec,
    )(q, k, v)


def _barriers(bar_sems):
    my = lax.axis_index("i")
    right = jnp.mod(my + 1, N_DEV)
    left = jnp.mod(my + 3, N_DEV)

    def entry():
        bsem = pltpu.get_barrier_semaphore()
        for nbr in (left, right):
            pl.semaphore_signal(bsem, inc=1, device_id=(nbr,),
                                device_id_type=pl.DeviceIdType.MESH)
        pl.semaphore_wait(bsem, 2)

    def exit_():
        for nbr in (left, right):
            pl.semaphore_signal(bar_sems.at[0], inc=1, device_id=(nbr,),
                                device_id_type=pl.DeviceIdType.MESH)
        pl.semaphore_wait(bar_sems.at[0], 2)

    return my, entry, exit_


def _ar1_ffn_ar2(ao, x, mod, wo_bf, W_ff1, W_ff2, B, S, D):
    R = B * S
    H = R // 2
    Ch = H // N_DEV
    Cs = Ch // 2

    def body(ao_ref, x_ref, mod_ref, wob_ref, wf1_ref, wf2_ref, out_ref,
             acc1, comm1, comm2, p2, acc2, wf1b, wf2b,
             send_sems, recv_sems, bar_sems):
        my, entry, exit_ = _barriers(bar_sems)
        right = jnp.mod(my + 1, N_DEV)
        left = jnp.mod(my + 3, N_DEV)
        tgt = (right, left)
        entry()

        def md(c):
            return jnp.mod(c + 8, N_DEV)

        def rcopy(src_ref, s_off, n, dst_ref, d_off, sem, d):
            return pltpu.make_async_remote_copy(
                src_ref=src_ref.at[pl.ds(s_off, n), :],
                dst_ref=(dst_ref if d_off is None
                         else dst_ref.at[pl.ds(d_off, n), :]),
                send_sem=send_sems.at[sem],
                recv_sem=recv_sems.at[sem],
                device_id=(tgt[d],),
                device_id_type=pl.DeviceIdType.MESH,
            )

        def sub_off(ck, d, u):
            return d * H + ck * Ch + u * Cs

        def rs1_send(s, d, u):
            ck = md(my - s) if d == 0 else md(my + s)
            rcopy(acc1, sub_off(ck, d, u), Cs,
                  comm1.at[s * 4 + d * 2 + u], None,
                  s * 4 + d * 2 + u, d).start()

        def rs1_wait_acc(s, d, u):
            ck = md(my - s) if d == 0 else md(my + s)
            rcopy(acc1, sub_off(ck, d, u), Cs,
                  comm1.at[s * 4 + d * 2 + u], None,
                  s * 4 + d * 2 + u, d).wait()
            rc = md(my - s - 1) if d == 0 else md(my + s + 1)
            o = sub_off(rc, d, u)
            acc1[pl.ds(o, Cs), :] = (
                acc1[pl.ds(o, Cs), :] + comm1[s * 4 + d * 2 + u])

        def proc_p1(b, ck):
            r = pl.ds(b * S + ck * Ch, Ch)
            acc1[r, :] = jnp.dot(ao_ref[r, :], wob_ref[:, :],
                                 preferred_element_type=F32).astype(BF16)

        def stage0(s, _):
            c0, c1 = md(my - s), md(my + s)
            proc_p1(0, c0)
            proc_p1(1, c1)

            @pl.when(s == 0)
            def _():
                for u in (0, 1):
                    for d in (0, 1):
                        rs1_send(0, d, u)
                wf1b[:, :] = wf1_ref[:, :].astype(BF16)
                wf2b[:, :] = wf2_ref[:, :].astype(BF16)

            @pl.when(s == 1)
            def _():
                for u in (0, 1):
                    for d in (0, 1):
                        rs1_wait_acc(0, d, u)
                        rs1_send(1, d, u)

            @pl.when(s == 2)
            def _():
                for u in (0, 1):
                    for d in (0, 1):
                        rs1_wait_acc(1, d, u)
                        rs1_send(2, d, u)

            @pl.when(s == 3)
            def _():
                for u in (0, 1):
                    for d in (0, 1):
                        rs1_wait_acc(2, d, u)

            return 0

        lax.fori_loop(0, N_DEV, stage0, 0)

        def proc(b, ck):
            rb = pl.ds(ck * Ch, Ch)
            r = pl.ds(b * S + ck * Ch, Ch)
            x1v = (x_ref[b, rb, :]
                   + (mod_ref[b, 2 * D:3 * D][None, :]).astype(BF16)
                   * acc1[r, :])
            m = jnp.mean(x1v, axis=1, keepdims=True, dtype=F32)
            c = x1v - m.astype(BF16)
            var = jnp.mean((c * c).astype(F32), axis=1, keepdims=True)
            xln2 = (c * lax.rsqrt(var + EPS).astype(BF16)
                    * (1.0 + mod_ref[b, 3 * D:4 * D][None, :]).astype(BF16)
                    + (mod_ref[b, 4 * D:5 * D][None, :]).astype(BF16))
            h = jnp.dot(xln2, wf1b[:, :], preferred_element_type=F32)
            h = (h / (1.0 + jnp.exp(-h))).astype(BF16)
            p2[r, :] = jnp.dot(h, wf2b[:, :],
                               preferred_element_type=F32).astype(BF16)

        def rs2_send(s, src, d, u):
            ck = md(my - s) if d == 0 else md(my + s)
            rcopy(src, sub_off(ck, d, u), Cs,
                  comm2.at[s * 4 + d * 2 + u], None,
                  24 + s * 4 + d * 2 + u, d).start()

        def rs2_wait_acc(s, src, d, u):
            ck = md(my - s) if d == 0 else md(my + s)
            rcopy(src, sub_off(ck, d, u), Cs,
                  comm2.at[s * 4 + d * 2 + u], None,
                  24 + s * 4 + d * 2 + u, d).wait()
            rc = md(my - s - 1) if d == 0 else md(my + s + 1)
            o = sub_off(rc, d, u)
            acc2[pl.ds(o, Cs), :] = (
                p2[pl.ds(o, Cs), :] + comm2[s * 4 + d * 2 + u])

        def ag1_io(t, d, u, start):
            ck = md(my + 1 - t) if d == 0 else md(my - 1 + t)
            o = sub_off(ck, d, u)
            r = rcopy(acc1, o, Cs, acc1, o, 12 + t * 4 + d * 2 + u, d)
            r.start() if start else r.wait()

        def stage1(t, _):
            for u in (0, 1):
                @pl.when(t > 0)
                def _():
                    for d in (0, 1):
                        ag1_io(t - 1, d, u, False)

                @pl.when(t < 3)
                def _():
                    for d in (0, 1):
                        ag1_io(t, d, u, True)

            proc(0, md(my + 1 - t))
            proc(1, md(my - 1 + t))

            @pl.when(t == 1)
            def _():
                for u in (0, 1):
                    for d in (0, 1):
                        rs2_send(0, p2, d, u)

            @pl.when(t == 2)
            def _():
                for u in (0, 1):
                    for d in (0, 1):
                        rs2_wait_acc(0, p2, d, u)
                        rs2_send(1, acc2, d, u)

            @pl.when(t == 3)
            def _():
                for u in (0, 1):
                    for d in (0, 1):
                        rs2_wait_acc(1, acc2, d, u)
                        rs2_send(2, acc2, d, u)
                for u in (0, 1):
                    for d in (0, 1):
                        rs2_wait_acc(2, acc2, d, u)

            return 0

        lax.fori_loop(0, N_DEV, stage1, 0)

        def ag2_io(t, d, u, start):
            ck = md(my + 1 - t) if d == 0 else md(my - 1 + t)
            o = sub_off(ck, d, u)
            r = rcopy(acc2, o, Cs, acc2, o, 36 + t * 4 + d * 2 + u, d)
            r.start() if start else r.wait()

        def stage2(t, _):
            for u in (0, 1):
                @pl.when(t > 0)
                def _():
                    for d in (0, 1):
                        ag2_io(t - 1, d, u, False)

                @pl.when(t < 3)
                def _():
                    for d in (0, 1):
                        ag2_io(t, d, u, True)

            for b, ck in ((0, md(my + 1 - t)), (1, md(my - 1 + t))):
                rb = pl.ds(ck * Ch, Ch)
                r = pl.ds(b * S + ck * Ch, Ch)
                out_ref[b, rb, :] = (
                    x_ref[b, rb, :].astype(F32)
                    + mod_ref[b, 2 * D:3 * D][None, :] * acc1[r, :].astype(F32)
                    + mod_ref[b, 5 * D:6 * D][None, :] * acc2[r, :].astype(F32))

            return 0

        lax.fori_loop(0, N_DEV, stage2, 0)
        exit_()

    return pl.pallas_call(
        body,
        out_shape=jax.ShapeDtypeStruct((B, S, D), F32),
        in_specs=_vmem(6),
        out_specs=_vmem()[0],
        scratch_shapes=[
            pltpu.VMEM((R, D), BF16),
            pltpu.VMEM((12, Cs, D), BF16),
            pltpu.VMEM((12, Cs, D), BF16),
            pltpu.VMEM((R, D), BF16),
            pltpu.VMEM((R, D), BF16),
            pltpu.VMEM((D, D), BF16),
            pltpu.VMEM((D, D), BF16),
            pltpu.SemaphoreType.DMA((48,)),
            pltpu.SemaphoreType.DMA((48,)),
            pltpu.SemaphoreType.REGULAR((1,)),
        ],
        compiler_params=pltpu.CompilerParams(
            collective_id=1, vmem_limit_bytes=44 * 1024 * 1024),
    )(ao, x, mod, wo_bf, W_ff1, W_ff2)


def kernel(x, Wq, Wk, Wv, Wo, t_emb, W_mod, W_ff1, W_ff2):
    B, S, D = x.shape
    Dh = 128
    H_loc = Wq.shape[1] // Dh

    mod, xm, xbf = _mod_ln(x, t_emb, W_mod, B, S, D)
    q, k, v, wo_bf = _qkv(xm, Wq, Wk, Wv, Wo)
    ao = _attention(q, k, v, B, S, H_loc, Dh)
    return _ar1_ffn_ar2(ao, xbf, mod, wo_bf, W_ff1, W_ff2, B, S, D)
